# baseline (speedup 1.0000x reference)
import dataclasses
import functools

import jax
import jax.numpy as jnp
from jax.experimental import pallas as pl
from jax.experimental.pallas import tpu as pltpu
from jax.experimental.pallas import tpu_sc as plsc

N = 10000
NP = 10240
E = 320000
D = 128
H = 512
T = 65536

_INTERP = False


def _prep_body(hist_ref, x_ref, dinv_ref, y0_ref):
    deg = 1.0 + hist_ref[0, :N, 0:1] + hist_ref[1, :N, 0:1]
    dinv = jax.lax.rsqrt(deg)
    dinv_ref[...] = dinv
    y0_ref[...] = x_ref[...] * dinv


def _prep_call(histT, x):
    return pl.pallas_call(
        _prep_body,
        out_shape=(
            jax.ShapeDtypeStruct((N, 1), jnp.float32),
            jax.ShapeDtypeStruct((N, D), jnp.float32),
        ),
        interpret=_INTERP,
    )(histT, x)


def _layer_body(nch, last, parts_ref, yprev_ref, dinv_ref, w_ref, b_ref,
                w4r_ref, o1_ref, o2_ref):
    bn = yprev_ref.shape[0]
    dinv = dinv_ref[...]
    acc = jnp.zeros((bn, H), jnp.float32)
    for q in range(nch):
        zq = (parts_ref[0, q] + parts_ref[1, q]
              + yprev_ref[:, q * 128:(q + 1) * 128]) * dinv
        acc = acc + jnp.dot(zq, w_ref[q * 128:(q + 1) * 128, :],
                            preferred_element_type=jnp.float32)
    h = jnp.tanh(acc + b_ref[...])
    if last:
        o1_ref[...] = h
        o2_ref[...] = h * w4r_ref[...]
    else:
        o1_ref[...] = h * dinv
        o2_ref[...] = h


def _layer_call(parts, yprev, dinv, w, b2d, w4r, last):
    nch = parts.shape[1]
    fin = nch * 128
    bn = 1000
    grid = (N // bn,)
    body = functools.partial(_layer_body, nch, last)
    return pl.pallas_call(
        body,
        grid=grid,
        in_specs=[
            pl.BlockSpec((2, nch, bn, 128), lambda i: (0, 0, i, 0)),
            pl.BlockSpec((bn, fin), lambda i: (i, 0)),
            pl.BlockSpec((bn, 1), lambda i: (i, 0)),
            pl.BlockSpec((fin, H), lambda i: (0, 0)),
            pl.BlockSpec((1, H), lambda i: (0, 0)),
            pl.BlockSpec((1, H), lambda i: (0, 0)),
        ],
        out_specs=(
            pl.BlockSpec((bn, H), lambda i: (i, 0)),
            pl.BlockSpec((bn, H), lambda i: (i, 0)),
        ),
        out_shape=(
            jax.ShapeDtypeStruct((N, H), jnp.float32),
            jax.ShapeDtypeStruct((N, H), jnp.float32),
        ),
        interpret=_INTERP,
    )(parts, yprev, dinv, w, b2d, w4r)


def _head_body(ga_ref, gb_ref, b4_ref, out_ref):
    prod = ga_ref[...] * gb_ref[...]
    out_ref[...] = jnp.sum(prod, axis=1, keepdims=True) + b4_ref[...]


def _head_call(ga, gb, b4_2d):
    bt = 2048
    return pl.pallas_call(
        _head_body,
        grid=(T // bt,),
        in_specs=[
            pl.BlockSpec((bt, H), lambda i: (i, 0)),
            pl.BlockSpec((bt, H), lambda i: (i, 0)),
            pl.BlockSpec((1, 1), lambda i: (0, 0)),
        ],
        out_specs=pl.BlockSpec((bt, 1), lambda i: (i, 0)),
        out_shape=jax.ShapeDtypeStruct((T, 1), jnp.float32),
        interpret=_INTERP,
    )(ga, gb, b4_2d)


_SC_MESH = plsc.VectorSubcoreMesh(core_axis_name="c", subcore_axis_name="s")

_SC_CP = pltpu.CompilerParams()
if "needs_layout_passes" in pltpu.CompilerParams.__dataclass_fields__:
    _SC_CP = dataclasses.replace(_SC_CP, needs_layout_passes=False)
EB = 80
EW = E // 32
NROW = NP // 16


def _deg_call(dst):
    RW = 128
    zeros = jnp.zeros((NROW, RW), jnp.float32)
    ones = jnp.ones((EB, RW), jnp.float32)

    @functools.partial(
        pl.kernel,
        out_type=jax.ShapeDtypeStruct((2, NP, RW), jnp.float32),
        mesh=_SC_MESH,
        scratch_types=[
            pltpu.VMEM((1, EB), jnp.int32),
            pltpu.VMEM((EB, RW), jnp.float32),
            pltpu.VMEM_SHARED((NP, RW), jnp.float32),
        ],
    )
    def k(dst_hbm, zeros_hbm, ones_hbm, out_hbm, dst_v, ones_v, shared):
        c = jax.lax.axis_index("c")
        s = jax.lax.axis_index("s")
        pltpu.sync_copy(zeros_hbm, shared.at[pl.ds(s * NROW, NROW)])
        pltpu.sync_copy(ones_hbm, ones_v)
        plsc.subcore_barrier()
        base = (c * 16 + s) * EW

        @pl.loop(0, EW // EB)
        def _(i):
            pltpu.sync_copy(dst_hbm.at[pl.ds(base + i * EB, EB)], dst_v.at[0])
            pltpu.sync_copy(ones_v, shared.at[dst_v.at[0]], add=True)

        plsc.subcore_barrier()
        pltpu.sync_copy(shared.at[pl.ds(s * NROW, NROW)],
                        out_hbm.at[c, pl.ds(s * NROW, NROW)])

    return k(dst, zeros, ones)


def _agg_call(table, src, dst, nch):
    zeros = jnp.zeros((NROW, 128), jnp.float32)
    half = E // 2

    @functools.partial(
        pl.kernel,
        out_type=jax.ShapeDtypeStruct((2, nch, NP, 128), jnp.float32),
        mesh=_SC_MESH,
        scratch_types=[
            pltpu.VMEM((EB,), jnp.int32),
            pltpu.VMEM((EB,), jnp.int32),
            pltpu.VMEM((1, EB), jnp.int32),
            pltpu.VMEM((EB, 128), jnp.float32),
            pltpu.VMEM_SHARED((NP, 128), jnp.float32),
        ],
    )
    def k(table_hbm, src_hbm, dst_hbm, zeros_hbm, out_hbm,
          src_v, idx_v, dst_v, rows_v, shared):
        c = jax.lax.axis_index("c")
        s = jax.lax.axis_index("s")
        base = c * half + s * (half // 16)

        for q in range(nch):
            pltpu.sync_copy(zeros_hbm, shared.at[pl.ds(s * NROW, NROW)])
            plsc.subcore_barrier()

            @pl.loop(0, half // 16 // EB)
            def _(i):
                eb = base + i * EB
                pltpu.sync_copy(src_hbm.at[pl.ds(eb, EB)], src_v)
                pltpu.sync_copy(dst_hbm.at[pl.ds(eb, EB)], dst_v.at[0])
                for t in range(EB // 16):
                    sl = pl.ds(t * 16, 16)
                    idx_v[sl] = src_v[sl] * nch + q
                pltpu.sync_copy(table_hbm.at[idx_v], rows_v)
                pltpu.sync_copy(rows_v, shared.at[dst_v.at[0]], add=True)

            plsc.subcore_barrier()
            pltpu.sync_copy(shared.at[pl.ds(s * NROW, NROW)],
                            out_hbm.at[c, q, pl.ds(s * NROW, NROW)])
            plsc.subcore_barrier()

    return k(table, src, dst, zeros)


def _head_sc_call(tid, pairs2, h3, h3w, b4b):
    BH = 32
    TW = T // 32
    NB = TW // BH

    @functools.partial(
        pl.kernel,
        out_type=jax.ShapeDtypeStruct((T,), jnp.float32),
        mesh=_SC_MESH,
        scratch_types=[
            pltpu.VMEM((BH,), jnp.int32),
            pltpu.VMEM((BH,), jnp.int32),
            pltpu.VMEM((BH, 128), jnp.int32),
            pltpu.VMEM((BH,), jnp.int32),
            pltpu.VMEM((BH,), jnp.int32),
            pltpu.VMEM((BH, H), jnp.float32),
            pltpu.VMEM((BH, H), jnp.float32),
            pltpu.VMEM((BH,), jnp.float32),
            pltpu.VMEM((16,), jnp.float32),
        ],
        compiler_params=_SC_CP,
    )
    def k(tid_hbm, pairs_hbm, h3_hbm, h3w_hbm, b4_hbm, out_hbm,
          tid_v, ridx_v, prow_v, a_v, b_v, ra_v, rb_v, out_v, b4_v):
        c = jax.lax.axis_index("c")
        s = jax.lax.axis_index("s")
        base = (c * 16 + s) * TW
        pltpu.sync_copy(b4_hbm, b4_v)
        iota = jax.lax.iota(jnp.int32, 16)

        @pl.loop(0, NB)
        def _(i):
            pltpu.sync_copy(tid_hbm.at[pl.ds(base + i * BH, BH)], tid_v)
            for g in range(BH // 16):
                sl = pl.ds(g * 16, 16)
                ridx_v[sl] = jax.lax.shift_right_logical(tid_v[sl], 6)
            pltpu.sync_copy(pairs_hbm.at[ridx_v], prow_v)
            for g in range(BH // 16):
                sl = pl.ds(g * 16, 16)
                rows = iota + g * 16
                lane = (tid_v[sl] & 63) * 2
                a_v[sl] = plsc.load_gather(prow_v, [rows, lane])
                b_v[sl] = plsc.load_gather(prow_v, [rows, lane + 1])
            pltpu.sync_copy(h3_hbm.at[a_v], ra_v)
            pltpu.sync_copy(h3w_hbm.at[b_v], rb_v)
            for g in range(BH // 16):
                rows = iota + g * 16
                acc = b4_v[...]

                def dotstep(j, acc):
                    cols = jnp.full((16,), j, jnp.int32)
                    va = plsc.load_gather(ra_v, [rows, cols])
                    vb = plsc.load_gather(rb_v, [rows, cols])
                    return acc + va * vb

                acc = jax.lax.fori_loop(0, H, dotstep, acc)
                out_v[pl.ds(g * 16, 16)] = acc
            pltpu.sync_copy(out_v, out_hbm.at[pl.ds(base + i * BH, BH)])

    return k(tid, pairs2, h3, h3w, b4b)


def _tmp_agg(y, src, dst, nch):
    s = jax.ops.segment_sum(y[src], dst, num_segments=N)
    sq = s.reshape(N, nch, 128).transpose(1, 0, 2)
    parts = jnp.zeros((2, nch, NP, 128), jnp.float32).at[0, :, :N].set(sq)
    return parts


def kernel(x, edge_index, train_edge_id, W1, b1, W2, b2, W3, b3, W4, b4):
    src = edge_index[0]
    dst = edge_index[1]

    hist = _deg_call(dst)
    dinv, y0 = _prep_call(hist, x)

    w4r = W4.reshape(1, H)
    b1r = b1.reshape(1, H)
    b2r = b2.reshape(1, H)
    b3r = b3.reshape(1, H)

    parts1 = _agg_call(y0, src, dst, 1)
    y1, _ = _layer_call(parts1, y0, dinv, W1, b1r, w4r, last=False)

    parts2 = _agg_call(y1.reshape(N * 4, 128), src, dst, 4)
    y2, _ = _layer_call(parts2, y1, dinv, W2, b2r, w4r, last=False)

    parts3 = _agg_call(y2.reshape(N * 4, 128), src, dst, 4)
    h3, h3w = _layer_call(parts3, y2, dinv, W3, b3r, w4r, last=True)

    pairs2 = jnp.stack([src, dst], axis=1).reshape(E // 64, 128)
    b4b = jnp.broadcast_to(b4, (16,)).astype(jnp.float32)
    out = _head_sc_call(train_edge_id, pairs2, h3, h3w, b4b)
    return out.reshape(T, 1)


# speedup vs baseline: 1.4366x; 1.4366x over previous
import dataclasses
import functools

import jax
import jax.numpy as jnp
from jax.experimental import pallas as pl
from jax.experimental.pallas import tpu as pltpu
from jax.experimental.pallas import tpu_sc as plsc

N = 10000
NP = 10000
E = 320000
D = 128
H = 512
T = 65536

_INTERP = False


def _prep_body(hist_ref, x_ref, dinv_ref, y0_ref):
    deg = 1.0 + hist_ref[0, :N, 0:1] + hist_ref[1, :N, 0:1]
    dinv = jax.lax.rsqrt(deg)
    dinv_ref[...] = dinv
    y0_ref[...] = x_ref[...] * dinv


def _prep_call(histT, x):
    return pl.pallas_call(
        _prep_body,
        out_shape=(
            jax.ShapeDtypeStruct((N, 1), jnp.float32),
            jax.ShapeDtypeStruct((N, D), jnp.float32),
        ),
        interpret=_INTERP,
    )(histT, x)


def _layer_body(nch, last, parts_ref, yprev_ref, dinv_ref, w_ref, b_ref,
                w4r_ref, o1_ref, o2_ref):
    bn = yprev_ref.shape[0]
    dinv = dinv_ref[...]
    acc = jnp.zeros((bn, H), jnp.float32)
    for q in range(nch):
        zq = (parts_ref[0, q] + parts_ref[1, q]
              + yprev_ref[:, q * 128:(q + 1) * 128]) * dinv
        acc = acc + jnp.dot(zq, w_ref[q * 128:(q + 1) * 128, :],
                            preferred_element_type=jnp.float32)
    h = jnp.tanh(acc + b_ref[...])
    if last:
        o1_ref[...] = h
        o2_ref[...] = h * w4r_ref[...]
    else:
        o1_ref[...] = h * dinv
        o2_ref[...] = h


def _layer_call(parts, yprev, dinv, w, b2d, w4r, last):
    nch = parts.shape[1]
    fin = nch * 128
    bn = 1000
    grid = (N // bn,)
    body = functools.partial(_layer_body, nch, last)
    return pl.pallas_call(
        body,
        grid=grid,
        in_specs=[
            pl.BlockSpec((2, nch, bn, 128), lambda i: (0, 0, i, 0)),
            pl.BlockSpec((bn, fin), lambda i: (i, 0)),
            pl.BlockSpec((bn, 1), lambda i: (i, 0)),
            pl.BlockSpec((fin, H), lambda i: (0, 0)),
            pl.BlockSpec((1, H), lambda i: (0, 0)),
            pl.BlockSpec((1, H), lambda i: (0, 0)),
        ],
        out_specs=(
            pl.BlockSpec((bn, H), lambda i: (i, 0)),
            pl.BlockSpec((bn, H), lambda i: (i, 0)),
        ),
        out_shape=(
            jax.ShapeDtypeStruct((N, H), jnp.float32),
            jax.ShapeDtypeStruct((N, H), jnp.float32),
        ),
        interpret=_INTERP,
    )(parts, yprev, dinv, w, b2d, w4r)


def _head_body(ga_ref, gb_ref, b4_ref, out_ref):
    prod = ga_ref[...] * gb_ref[...]
    out_ref[...] = jnp.sum(prod, axis=1, keepdims=True) + b4_ref[...]


def _head_call(ga, gb, b4_2d):
    bt = 2048
    return pl.pallas_call(
        _head_body,
        grid=(T // bt,),
        in_specs=[
            pl.BlockSpec((bt, H), lambda i: (i, 0)),
            pl.BlockSpec((bt, H), lambda i: (i, 0)),
            pl.BlockSpec((1, 1), lambda i: (0, 0)),
        ],
        out_specs=pl.BlockSpec((bt, 1), lambda i: (i, 0)),
        out_shape=jax.ShapeDtypeStruct((T, 1), jnp.float32),
        interpret=_INTERP,
    )(ga, gb, b4_2d)


_SC_MESH = plsc.VectorSubcoreMesh(core_axis_name="c", subcore_axis_name="s")

_SC_CP = pltpu.CompilerParams()
if "needs_layout_passes" in pltpu.CompilerParams.__dataclass_fields__:
    _SC_CP = dataclasses.replace(_SC_CP, needs_layout_passes=False)

NR0 = 624
NRL = NP - 15 * NR0


def _ranged_copy(s, mk_src, mk_dst):
    @pl.when(s < 15)
    def _():
        pltpu.sync_copy(mk_src(s * NR0, NR0), mk_dst(s * NR0, NR0))

    @pl.when(s == 15)
    def _():
        pltpu.sync_copy(mk_src(15 * NR0, NRL), mk_dst(15 * NR0, NRL))
EB = 80
EW = E // 32


def _deg_call(dst):
    RW = 128
    zeros = jnp.zeros((NRL, RW), jnp.float32)
    ones = jnp.ones((EB, RW), jnp.float32)

    @functools.partial(
        pl.kernel,
        out_type=jax.ShapeDtypeStruct((2, NP, RW), jnp.float32),
        mesh=_SC_MESH,
        scratch_types=[
            pltpu.VMEM((1, EB), jnp.int32),
            pltpu.VMEM((EB, RW), jnp.float32),
            pltpu.VMEM_SHARED((NP, RW), jnp.float32),
        ],
    )
    def k(dst_hbm, zeros_hbm, ones_hbm, out_hbm, dst_v, ones_v, shared):
        c = jax.lax.axis_index("c")
        s = jax.lax.axis_index("s")
        _ranged_copy(s, lambda o, L: zeros_hbm.at[pl.ds(0, L)],
                     lambda o, L: shared.at[pl.ds(o, L)])
        pltpu.sync_copy(ones_hbm, ones_v)
        plsc.subcore_barrier()
        base = (c * 16 + s) * EW

        @pl.loop(0, EW // EB)
        def _(i):
            pltpu.sync_copy(dst_hbm.at[pl.ds(base + i * EB, EB)], dst_v.at[0])
            pltpu.sync_copy(ones_v, shared.at[dst_v.at[0]], add=True)

        plsc.subcore_barrier()
        _ranged_copy(s, lambda o, L: shared.at[pl.ds(o, L)],
                     lambda o, L: out_hbm.at[c, pl.ds(o, L)])

    return k(dst, zeros, ones)


EW2 = E // 32
NBLK = EW2 // EB


def _agg_call(table, src, dst, nch):
    zeros = jnp.zeros((NRL, 128), jnp.float32)
    half = E // 2

    @functools.partial(
        pl.kernel,
        out_type=jax.ShapeDtypeStruct((2, nch, NP, 128), jnp.float32),
        mesh=_SC_MESH,
        scratch_types=[
            pltpu.VMEM((EW2,), jnp.int32),
            pltpu.VMEM((EW2,), jnp.int32),
            pltpu.VMEM((EW2,), jnp.int32),
            pltpu.VMEM((2, EB), jnp.int32),
            pltpu.VMEM((2, EB, 128), jnp.float32),
            pltpu.VMEM_SHARED((NP, 128), jnp.float32),
            pltpu.SemaphoreType.DMA,
            pltpu.SemaphoreType.DMA,
            pltpu.SemaphoreType.DMA,
            pltpu.SemaphoreType.DMA,
        ],
    )
    def k(table_hbm, src_hbm, dst_hbm, zeros_hbm, out_hbm,
          srcn_v, dst_v, idx_v, dstb_v, rows_v, shared, gs0, gs1, ss0, ss1):
        c = jax.lax.axis_index("c")
        s = jax.lax.axis_index("s")
        base = c * half + s * EW2
        gsems = (gs0, gs1)
        ssems = (ss0, ss1)

        pltpu.sync_copy(src_hbm.at[pl.ds(base, EW2)], srcn_v)
        pltpu.sync_copy(dst_hbm.at[pl.ds(base, EW2)], dst_v)

        if nch > 1:
            @pl.loop(0, EW2 // 16)
            def _(t):
                sl = pl.ds(t * 16, 16)
                srcn_v[sl] = srcn_v[sl] * nch

        def gather_start(p, i):
            pltpu.async_copy(table_hbm.at[idx_v.at[pl.ds(i * EB, EB)]],
                             rows_v.at[p], gsems[p])

        def gather_wait(p, i):
            pltpu.make_async_copy(table_hbm.at[idx_v.at[pl.ds(i * EB, EB)]],
                                  rows_v.at[p], gsems[p]).wait()

        def load_dstb(p, i):
            for t in range(EB // 16):
                sl = pl.ds(t * 16, 16)
                dstb_v[p, sl] = dst_v[pl.ds(i * EB + t * 16, 16)]

        def scat_start(p):
            pltpu.async_copy(rows_v.at[p], shared.at[dstb_v.at[p]], ssems[p],
                             add=True)

        def scat_wait(p):
            pltpu.make_async_copy(rows_v.at[p], shared.at[dstb_v.at[p]],
                                  ssems[p]).wait()

        for q in range(nch):
            @pl.loop(0, EW2 // 16)
            def _(t):
                sl = pl.ds(t * 16, 16)
                idx_v[sl] = srcn_v[sl] + q
            _ranged_copy(s, lambda o, L: zeros_hbm.at[pl.ds(0, L)],
                         lambda o, L: shared.at[pl.ds(o, L)])
            plsc.subcore_barrier()

            gather_start(0, 0)

            @pl.loop(0, (NBLK - 1) // 2)
            def _(j):
                for p in (0, 1):
                    i = 2 * j + p
                    gather_wait(p, i)
                    load_dstb(p, i)
                    scat_start(p)
                    if p == 1:
                        scat_wait(0)
                    else:
                        @pl.when(j > 0)
                        def _():
                            scat_wait(1)
                    gather_start(p ^ 1, i + 1)

            last = NBLK - 1
            lp = last % 2
            gather_wait(lp, last)
            load_dstb(lp, last)
            scat_start(lp)
            scat_wait(lp ^ 1)
            scat_wait(lp)

            plsc.subcore_barrier()
            _ranged_copy(s, lambda o, L: shared.at[pl.ds(o, L)],
                         lambda o, L: out_hbm.at[c, q, pl.ds(o, L)])
            plsc.subcore_barrier()

    return k(table, src, dst, zeros)


def _head_sc_call(tid, pairs2, h3, h3w, b4b):
    BH = 32
    TW = T // 32
    NB = TW // BH

    @functools.partial(
        pl.kernel,
        out_type=jax.ShapeDtypeStruct((T,), jnp.float32),
        mesh=_SC_MESH,
        scratch_types=[
            pltpu.VMEM((BH,), jnp.int32),
            pltpu.VMEM((BH,), jnp.int32),
            pltpu.VMEM((BH, 128), jnp.int32),
            pltpu.VMEM((BH,), jnp.int32),
            pltpu.VMEM((BH,), jnp.int32),
            pltpu.VMEM((BH, H), jnp.float32),
            pltpu.VMEM((BH, H), jnp.float32),
            pltpu.VMEM((BH,), jnp.float32),
            pltpu.VMEM((16,), jnp.float32),
        ],
        compiler_params=_SC_CP,
    )
    def k(tid_hbm, pairs_hbm, h3_hbm, h3w_hbm, b4_hbm, out_hbm,
          tid_v, ridx_v, prow_v, a_v, b_v, ra_v, rb_v, out_v, b4_v):
        c = jax.lax.axis_index("c")
        s = jax.lax.axis_index("s")
        base = (c * 16 + s) * TW
        pltpu.sync_copy(b4_hbm, b4_v)
        iota = jax.lax.iota(jnp.int32, 16)

        @pl.loop(0, NB)
        def _(i):
            pltpu.sync_copy(tid_hbm.at[pl.ds(base + i * BH, BH)], tid_v)
            for g in range(BH // 16):
                sl = pl.ds(g * 16, 16)
                ridx_v[sl] = jax.lax.shift_right_logical(tid_v[sl], 6)
            pltpu.sync_copy(pairs_hbm.at[ridx_v], prow_v)
            for g in range(BH // 16):
                sl = pl.ds(g * 16, 16)
                rows = iota + g * 16
                lane = (tid_v[sl] & 63) * 2
                a_v[sl] = plsc.load_gather(prow_v, [rows, lane])
                b_v[sl] = plsc.load_gather(prow_v, [rows, lane + 1])
            pltpu.sync_copy(h3_hbm.at[a_v], ra_v)
            pltpu.sync_copy(h3w_hbm.at[b_v], rb_v)
            for g in range(BH // 16):
                rows = iota + g * 16
                acc = b4_v[...]

                def dotstep(j, acc):
                    cols = jnp.full((16,), j, jnp.int32)
                    va = plsc.load_gather(ra_v, [rows, cols])
                    vb = plsc.load_gather(rb_v, [rows, cols])
                    return acc + va * vb

                acc = jax.lax.fori_loop(0, H, dotstep, acc)
                out_v[pl.ds(g * 16, 16)] = acc
            pltpu.sync_copy(out_v, out_hbm.at[pl.ds(base + i * BH, BH)])

    return k(tid, pairs2, h3, h3w, b4b)


def _tmp_agg(y, src, dst, nch):
    s = jax.ops.segment_sum(y[src], dst, num_segments=N)
    sq = s.reshape(N, nch, 128).transpose(1, 0, 2)
    parts = jnp.zeros((2, nch, NP, 128), jnp.float32).at[0, :, :N].set(sq)
    return parts


def kernel(x, edge_index, train_edge_id, W1, b1, W2, b2, W3, b3, W4, b4):
    src = edge_index[0]
    dst = edge_index[1]

    hist = _deg_call(dst)
    dinv, y0 = _prep_call(hist, x)

    w4r = W4.reshape(1, H)
    b1r = b1.reshape(1, H)
    b2r = b2.reshape(1, H)
    b3r = b3.reshape(1, H)

    parts1 = _agg_call(y0, src, dst, 1)
    y1, _ = _layer_call(parts1, y0, dinv, W1, b1r, w4r, last=False)

    parts2 = _agg_call(y1.reshape(N * 4, 128), src, dst, 4)
    y2, _ = _layer_call(parts2, y1, dinv, W2, b2r, w4r, last=False)

    parts3 = _agg_call(y2.reshape(N * 4, 128), src, dst, 4)
    h3, h3w = _layer_call(parts3, y2, dinv, W3, b3r, w4r, last=True)

    pairs2 = jnp.stack([src, dst], axis=1).reshape(E // 64, 128)
    b4b = jnp.broadcast_to(b4, (16,)).astype(jnp.float32)
    out = _head_sc_call(train_edge_id, pairs2, h3, h3w, b4b)
    return out.reshape(T, 1)


# speedup vs baseline: 2.1386x; 1.4887x over previous
import dataclasses
import functools

import jax
import jax.numpy as jnp
from jax.experimental import pallas as pl
from jax.experimental.pallas import tpu as pltpu
from jax.experimental.pallas import tpu_sc as plsc

N = 10000
NP = 10000
E = 320000
D = 128
H = 512
T = 65536

_INTERP = False


def _prep_body(hist_ref, x_ref, dinv_ref, y0_ref):
    deg = 1.0 + hist_ref[0, :N, 0:1] + hist_ref[1, :N, 0:1]
    dinv = jax.lax.rsqrt(deg)
    dinv_ref[...] = dinv
    y0_ref[...] = x_ref[...] * dinv


def _prep_call(histT, x):
    return pl.pallas_call(
        _prep_body,
        out_shape=(
            jax.ShapeDtypeStruct((N, 1), jnp.float32),
            jax.ShapeDtypeStruct((N, D), jnp.float32),
        ),
        interpret=_INTERP,
    )(histT, x)


def _layer_body(nch, last, parts_ref, yprev_ref, dinv_ref, w_ref, b_ref,
                w4r_ref, o1_ref, o2_ref):
    bn = yprev_ref.shape[0]
    dinv = dinv_ref[...]
    acc = jnp.zeros((bn, H), jnp.float32)
    for q in range(nch):
        zq = (parts_ref[0, q] + parts_ref[1, q]
              + yprev_ref[:, q * 128:(q + 1) * 128]) * dinv
        acc = acc + jnp.dot(zq, w_ref[q * 128:(q + 1) * 128, :],
                            preferred_element_type=jnp.float32)
    h = jnp.tanh(acc + b_ref[...])
    if last:
        o1_ref[...] = h
        o2_ref[...] = h * w4r_ref[...]
    else:
        o1_ref[...] = h * dinv
        o2_ref[...] = h


def _layer_call(parts, yprev, dinv, w, b2d, w4r, last):
    nch = parts.shape[1]
    fin = nch * 128
    bn = 1000
    grid = (N // bn,)
    body = functools.partial(_layer_body, nch, last)
    return pl.pallas_call(
        body,
        grid=grid,
        in_specs=[
            pl.BlockSpec((2, nch, bn, 128), lambda i: (0, 0, i, 0)),
            pl.BlockSpec((bn, fin), lambda i: (i, 0)),
            pl.BlockSpec((bn, 1), lambda i: (i, 0)),
            pl.BlockSpec((fin, H), lambda i: (0, 0)),
            pl.BlockSpec((1, H), lambda i: (0, 0)),
            pl.BlockSpec((1, H), lambda i: (0, 0)),
        ],
        out_specs=(
            pl.BlockSpec((bn, H), lambda i: (i, 0)),
            pl.BlockSpec((bn, H), lambda i: (i, 0)),
        ),
        out_shape=(
            jax.ShapeDtypeStruct((N, H), jnp.float32),
            jax.ShapeDtypeStruct((N, H), jnp.float32),
        ),
        interpret=_INTERP,
    )(parts, yprev, dinv, w, b2d, w4r)


def _headsum_body(p16_ref, b4_ref, out_ref):
    out_ref[...] = jnp.sum(p16_ref[...], axis=1, keepdims=True) + b4_ref[...]


def _headsum_call(p16, b4_2d):
    bt = 2048
    return pl.pallas_call(
        _headsum_body,
        grid=(T // bt,),
        in_specs=[
            pl.BlockSpec((bt, 16), lambda i: (i, 0)),
            pl.BlockSpec((1, 1), lambda i: (0, 0)),
        ],
        out_specs=pl.BlockSpec((bt, 1), lambda i: (i, 0)),
        out_shape=jax.ShapeDtypeStruct((T, 1), jnp.float32),
        interpret=_INTERP,
    )(p16, b4_2d)


_SC_MESH = plsc.VectorSubcoreMesh(core_axis_name="c", subcore_axis_name="s")

_SC_CP = pltpu.CompilerParams()
if "needs_layout_passes" in pltpu.CompilerParams.__dataclass_fields__:
    _SC_CP = dataclasses.replace(_SC_CP, needs_layout_passes=False)

NR0 = 624
NRL = NP - 15 * NR0


def _ranged_copy(s, mk_src, mk_dst):
    @pl.when(s < 15)
    def _():
        pltpu.sync_copy(mk_src(s * NR0, NR0), mk_dst(s * NR0, NR0))

    @pl.when(s == 15)
    def _():
        pltpu.sync_copy(mk_src(15 * NR0, NRL), mk_dst(15 * NR0, NRL))
EB = 80
EW = E // 32


def _deg_call(dst):
    RW = 128
    zeros = jnp.zeros((NRL, RW), jnp.float32)
    ones = jnp.ones((EB, RW), jnp.float32)

    @functools.partial(
        pl.kernel,
        out_type=jax.ShapeDtypeStruct((2, NP, RW), jnp.float32),
        mesh=_SC_MESH,
        scratch_types=[
            pltpu.VMEM((1, EB), jnp.int32),
            pltpu.VMEM((EB, RW), jnp.float32),
            pltpu.VMEM_SHARED((NP, RW), jnp.float32),
        ],
    )
    def k(dst_hbm, zeros_hbm, ones_hbm, out_hbm, dst_v, ones_v, shared):
        c = jax.lax.axis_index("c")
        s = jax.lax.axis_index("s")
        _ranged_copy(s, lambda o, L: zeros_hbm.at[pl.ds(0, L)],
                     lambda o, L: shared.at[pl.ds(o, L)])
        pltpu.sync_copy(ones_hbm, ones_v)
        plsc.subcore_barrier()
        base = (c * 16 + s) * EW

        @pl.loop(0, EW // EB)
        def _(i):
            pltpu.sync_copy(dst_hbm.at[pl.ds(base + i * EB, EB)], dst_v.at[0])
            pltpu.sync_copy(ones_v, shared.at[dst_v.at[0]], add=True)

        plsc.subcore_barrier()
        _ranged_copy(s, lambda o, L: shared.at[pl.ds(o, L)],
                     lambda o, L: out_hbm.at[c, pl.ds(o, L)])

    return k(dst, zeros, ones)


EW2 = E // 32
NBLK = EW2 // EB


def _agg_call(table, src, dst, nch):
    zeros = jnp.zeros((NRL, 128), jnp.float32)
    half = E // 2

    @functools.partial(
        pl.kernel,
        out_type=jax.ShapeDtypeStruct((2, nch, NP, 128), jnp.float32),
        mesh=_SC_MESH,
        scratch_types=[
            pltpu.VMEM((EW2,), jnp.int32),
            pltpu.VMEM((EW2,), jnp.int32),
            pltpu.VMEM((EW2,), jnp.int32),
            pltpu.VMEM((2, EB), jnp.int32),
            pltpu.VMEM((2, EB, 128), jnp.float32),
            pltpu.VMEM_SHARED((NP, 128), jnp.float32),
            pltpu.SemaphoreType.DMA,
            pltpu.SemaphoreType.DMA,
            pltpu.SemaphoreType.DMA,
            pltpu.SemaphoreType.DMA,
        ],
    )
    def k(table_hbm, src_hbm, dst_hbm, zeros_hbm, out_hbm,
          srcn_v, dst_v, idx_v, dstb_v, rows_v, shared, gs0, gs1, ss0, ss1):
        c = jax.lax.axis_index("c")
        s = jax.lax.axis_index("s")
        base = c * half + s * EW2
        gsems = (gs0, gs1)
        ssems = (ss0, ss1)

        pltpu.sync_copy(src_hbm.at[pl.ds(base, EW2)], srcn_v)
        pltpu.sync_copy(dst_hbm.at[pl.ds(base, EW2)], dst_v)

        if nch > 1:
            @pl.loop(0, EW2 // 16)
            def _(t):
                sl = pl.ds(t * 16, 16)
                srcn_v[sl] = srcn_v[sl] * nch

        def gather_start(p, i):
            pltpu.async_copy(table_hbm.at[idx_v.at[pl.ds(i * EB, EB)]],
                             rows_v.at[p], gsems[p])

        def gather_wait(p, i):
            pltpu.make_async_copy(table_hbm.at[idx_v.at[pl.ds(i * EB, EB)]],
                                  rows_v.at[p], gsems[p]).wait()

        def load_dstb(p, i):
            for t in range(EB // 16):
                sl = pl.ds(t * 16, 16)
                dstb_v[p, sl] = dst_v[pl.ds(i * EB + t * 16, 16)]

        def scat_start(p):
            pltpu.async_copy(rows_v.at[p], shared.at[dstb_v.at[p]], ssems[p],
                             add=True)

        def scat_wait(p):
            pltpu.make_async_copy(rows_v.at[p], shared.at[dstb_v.at[p]],
                                  ssems[p]).wait()

        for q in range(nch):
            @pl.loop(0, EW2 // 16)
            def _(t):
                sl = pl.ds(t * 16, 16)
                idx_v[sl] = srcn_v[sl] + q
            _ranged_copy(s, lambda o, L: zeros_hbm.at[pl.ds(0, L)],
                         lambda o, L: shared.at[pl.ds(o, L)])
            plsc.subcore_barrier()

            gather_start(0, 0)

            @pl.loop(0, (NBLK - 1) // 2)
            def _(j):
                for p in (0, 1):
                    i = 2 * j + p
                    gather_wait(p, i)
                    load_dstb(p, i)
                    scat_start(p)
                    if p == 1:
                        scat_wait(0)
                    else:
                        @pl.when(j > 0)
                        def _():
                            scat_wait(1)
                    gather_start(p ^ 1, i + 1)

            last = NBLK - 1
            lp = last % 2
            gather_wait(lp, last)
            load_dstb(lp, last)
            scat_start(lp)
            scat_wait(lp ^ 1)
            scat_wait(lp)

            plsc.subcore_barrier()
            _ranged_copy(s, lambda o, L: shared.at[pl.ds(o, L)],
                         lambda o, L: out_hbm.at[c, q, pl.ds(o, L)])
            plsc.subcore_barrier()

    return k(table, src, dst, zeros)


def _head_sc_call(tid, pairs2, h3, h3w):
    BH = 32
    TW = T // 32
    NB = TW // BH

    @functools.partial(
        pl.kernel,
        out_type=jax.ShapeDtypeStruct((T, 16), jnp.float32),
        mesh=_SC_MESH,
        scratch_types=[
            pltpu.VMEM((TW,), jnp.int32),
            pltpu.VMEM((TW,), jnp.int32),
            pltpu.VMEM((BH, 128), jnp.int32),
            pltpu.VMEM((2, BH), jnp.int32),
            pltpu.VMEM((2, BH), jnp.int32),
            pltpu.VMEM((2, BH, H), jnp.float32),
            pltpu.VMEM((2, BH, H), jnp.float32),
            pltpu.VMEM((BH, 16), jnp.float32),
            pltpu.SemaphoreType.DMA,
            pltpu.SemaphoreType.DMA,
        ],
        compiler_params=_SC_CP,
    )
    def k(tid_hbm, pairs_hbm, h3_hbm, h3w_hbm, out_hbm,
          tid_v, ridx_v, prow_v, a_v, b_v, ra_v, rb_v, o16_v, rs0, rs1):
        c = jax.lax.axis_index("c")
        s = jax.lax.axis_index("s")
        base = (c * 16 + s) * TW
        rsems = (rs0, rs1)
        iota = jax.lax.iota(jnp.int32, 16)

        pltpu.sync_copy(tid_hbm.at[pl.ds(base, TW)], tid_v)

        @pl.loop(0, TW // 16)
        def _(t):
            sl = pl.ds(t * 16, 16)
            ridx_v[sl] = jax.lax.shift_right_logical(tid_v[sl], 6)

        def stage_a(p, i):
            pltpu.sync_copy(pairs_hbm.at[ridx_v.at[pl.ds(i * BH, BH)]],
                            prow_v)
            for g in range(BH // 16):
                sl = pl.ds(g * 16, 16)
                rows = iota + g * 16
                lane = (tid_v[pl.ds(i * BH + g * 16, 16)] & 63) * 2
                a_v[p, sl] = plsc.load_gather(prow_v, [rows, lane])
                b_v[p, sl] = plsc.load_gather(prow_v, [rows, lane + 1])
            pltpu.async_copy(h3_hbm.at[a_v.at[p]], ra_v.at[p], rsems[p])
            pltpu.async_copy(h3w_hbm.at[b_v.at[p]], rb_v.at[p], rsems[p])

        def rows_wait(p):
            pltpu.make_async_copy(h3_hbm.at[a_v.at[p]], ra_v.at[p],
                                  rsems[p]).wait()
            pltpu.make_async_copy(h3w_hbm.at[b_v.at[p]], rb_v.at[p],
                                  rsems[p]).wait()

        stage_a(0, 0)

        @pl.loop(0, NB // 2)
        def _(j):
            for p in (0, 1):
                i = 2 * j + p
                rows_wait(p)

                @pl.when(i < NB - 1)
                def _():
                    stage_a(p ^ 1, i + 1)

                def edge_dot(e, _):
                    acc = jnp.zeros((16,), jnp.float32)
                    for t in range(H // 16):
                        sl = pl.ds(t * 16, 16)
                        acc = acc + ra_v[p, e, sl] * rb_v[p, e, sl]
                    o16_v[e, :] = acc
                    return 0

                jax.lax.fori_loop(0, BH, edge_dot, 0)
                pltpu.sync_copy(o16_v,
                                out_hbm.at[pl.ds(base + i * BH, BH)])

    return k(tid, pairs2, h3, h3w)


def _tmp_agg(y, src, dst, nch):
    s = jax.ops.segment_sum(y[src], dst, num_segments=N)
    sq = s.reshape(N, nch, 128).transpose(1, 0, 2)
    parts = jnp.zeros((2, nch, NP, 128), jnp.float32).at[0, :, :N].set(sq)
    return parts


def kernel(x, edge_index, train_edge_id, W1, b1, W2, b2, W3, b3, W4, b4):
    src = edge_index[0]
    dst = edge_index[1]

    hist = _deg_call(dst)
    dinv, y0 = _prep_call(hist, x)

    w4r = W4.reshape(1, H)
    b1r = b1.reshape(1, H)
    b2r = b2.reshape(1, H)
    b3r = b3.reshape(1, H)

    parts1 = _agg_call(y0, src, dst, 1)
    y1, _ = _layer_call(parts1, y0, dinv, W1, b1r, w4r, last=False)

    parts2 = _agg_call(y1.reshape(N * 4, 128), src, dst, 4)
    y2, _ = _layer_call(parts2, y1, dinv, W2, b2r, w4r, last=False)

    parts3 = _agg_call(y2.reshape(N * 4, 128), src, dst, 4)
    h3, h3w = _layer_call(parts3, y2, dinv, W3, b3r, w4r, last=True)

    pairs2 = jnp.stack([src, dst], axis=1).reshape(E // 64, 128)
    p16 = _head_sc_call(train_edge_id, pairs2, h3, h3w)
    return _headsum_call(p16, b4.reshape(1, 1))


# speedup vs baseline: 2.2637x; 1.0585x over previous
import dataclasses
import functools

import jax
import jax.numpy as jnp
from jax.experimental import pallas as pl
from jax.experimental.pallas import tpu as pltpu
from jax.experimental.pallas import tpu_sc as plsc

N = 10000
NP = 10000
E = 320000
D = 128
H = 512
T = 65536

_INTERP = False


def _prep_body(hist_ref, x_ref, dinv_ref, y0_ref):
    deg = 1.0 + hist_ref[0, :N, 0:1] + hist_ref[1, :N, 0:1]
    dinv = jax.lax.rsqrt(deg)
    dinv_ref[...] = dinv
    y0_ref[...] = x_ref[...] * dinv


def _prep_call(histT, x):
    return pl.pallas_call(
        _prep_body,
        out_shape=(
            jax.ShapeDtypeStruct((N, 1), jnp.float32),
            jax.ShapeDtypeStruct((N, D), jnp.float32),
        ),
        interpret=_INTERP,
    )(histT, x)


def _layer_body(nch, last, parts_ref, yprev_ref, dinv_ref, w_ref, b_ref,
                w4r_ref, o1_ref, o2_ref):
    bn = yprev_ref.shape[0]
    dinv = dinv_ref[...]
    acc = jnp.zeros((bn, H), jnp.float32)
    for q in range(nch):
        zq = (parts_ref[0, q] + parts_ref[1, q]
              + yprev_ref[:, q * 128:(q + 1) * 128]) * dinv
        acc = acc + jnp.dot(zq, w_ref[q * 128:(q + 1) * 128, :],
                            preferred_element_type=jnp.float32)
    h = jnp.tanh(acc + b_ref[...])
    if last:
        o1_ref[...] = h
        o2_ref[...] = h * w4r_ref[...]
    else:
        o1_ref[...] = h * dinv
        o2_ref[...] = h


def _layer_call(parts, yprev, dinv, w, b2d, w4r, last):
    nch = parts.shape[1]
    fin = nch * 128
    bn = 1000
    grid = (N // bn,)
    body = functools.partial(_layer_body, nch, last)
    return pl.pallas_call(
        body,
        grid=grid,
        in_specs=[
            pl.BlockSpec((2, nch, bn, 128), lambda i: (0, 0, i, 0)),
            pl.BlockSpec((bn, fin), lambda i: (i, 0)),
            pl.BlockSpec((bn, 1), lambda i: (i, 0)),
            pl.BlockSpec((fin, H), lambda i: (0, 0)),
            pl.BlockSpec((1, H), lambda i: (0, 0)),
            pl.BlockSpec((1, H), lambda i: (0, 0)),
        ],
        out_specs=(
            pl.BlockSpec((bn, H), lambda i: (i, 0)),
            pl.BlockSpec((bn, H), lambda i: (i, 0)),
        ),
        out_shape=(
            jax.ShapeDtypeStruct((N, H), jnp.float32),
            jax.ShapeDtypeStruct((N, H), jnp.float32),
        ),
        interpret=_INTERP,
    )(parts, yprev, dinv, w, b2d, w4r)


def _headsum_body(p16_ref, b4_ref, out_ref):
    out_ref[...] = jnp.sum(p16_ref[...], axis=1, keepdims=True) + b4_ref[...]


def _headsum_call(p16, b4_2d):
    bt = 2048
    return pl.pallas_call(
        _headsum_body,
        grid=(T // bt,),
        in_specs=[
            pl.BlockSpec((bt, 16), lambda i: (i, 0)),
            pl.BlockSpec((1, 1), lambda i: (0, 0)),
        ],
        out_specs=pl.BlockSpec((bt, 1), lambda i: (i, 0)),
        out_shape=jax.ShapeDtypeStruct((T, 1), jnp.float32),
        interpret=_INTERP,
    )(p16, b4_2d)


_SC_MESH = plsc.VectorSubcoreMesh(core_axis_name="c", subcore_axis_name="s")

_SC_CP = pltpu.CompilerParams()
if "needs_layout_passes" in pltpu.CompilerParams.__dataclass_fields__:
    _SC_CP = dataclasses.replace(_SC_CP, needs_layout_passes=False)

NR0 = 624
NRL = NP - 15 * NR0


def _ranged_copy(s, mk_src, mk_dst):
    @pl.when(s < 15)
    def _():
        pltpu.sync_copy(mk_src(s * NR0, NR0), mk_dst(s * NR0, NR0))

    @pl.when(s == 15)
    def _():
        pltpu.sync_copy(mk_src(15 * NR0, NRL), mk_dst(15 * NR0, NRL))
EB = 80
EW = E // 32


def _deg_call(dst):
    RW = 128
    zeros = jnp.zeros((NRL, RW), jnp.float32)
    ones = jnp.ones((EB, RW), jnp.float32)

    @functools.partial(
        pl.kernel,
        out_type=jax.ShapeDtypeStruct((2, NP, RW), jnp.float32),
        mesh=_SC_MESH,
        scratch_types=[
            pltpu.VMEM((1, EB), jnp.int32),
            pltpu.VMEM((EB, RW), jnp.float32),
            pltpu.VMEM_SHARED((NP, RW), jnp.float32),
        ],
    )
    def k(dst_hbm, zeros_hbm, ones_hbm, out_hbm, dst_v, ones_v, shared):
        c = jax.lax.axis_index("c")
        s = jax.lax.axis_index("s")
        _ranged_copy(s, lambda o, L: zeros_hbm.at[pl.ds(0, L)],
                     lambda o, L: shared.at[pl.ds(o, L)])
        pltpu.sync_copy(ones_hbm, ones_v)
        plsc.subcore_barrier()
        base = (c * 16 + s) * EW

        @pl.loop(0, EW // EB)
        def _(i):
            pltpu.sync_copy(dst_hbm.at[pl.ds(base + i * EB, EB)], dst_v.at[0])
            pltpu.sync_copy(ones_v, shared.at[dst_v.at[0]], add=True)

        plsc.subcore_barrier()
        _ranged_copy(s, lambda o, L: shared.at[pl.ds(o, L)],
                     lambda o, L: out_hbm.at[c, pl.ds(o, L)])

    return k(dst, zeros, ones)


EW2 = E // 32
EBA = 96
NBLKF = EW2 // EBA
TAIL = EW2 - NBLKF * EBA
TOFF = NBLKF * EBA


def _agg_call(table, src, dst, nch):
    zeros = jnp.zeros((NRL, 128), jnp.float32)
    half = E // 2

    @functools.partial(
        pl.kernel,
        out_type=jax.ShapeDtypeStruct((2, nch, NP, 128), jnp.float32),
        mesh=_SC_MESH,
        scratch_types=[
            pltpu.VMEM((EW2,), jnp.int32),
            pltpu.VMEM((EW2,), jnp.int32),
            pltpu.VMEM((2, EBA), jnp.int32),
            pltpu.VMEM((2, EBA, 128), jnp.float32),
            pltpu.VMEM((1, TAIL), jnp.int32),
            pltpu.VMEM((TAIL, 128), jnp.float32),
            pltpu.VMEM_SHARED((NP, 128), jnp.float32),
            pltpu.SemaphoreType.DMA,
            pltpu.SemaphoreType.DMA,
            pltpu.SemaphoreType.DMA,
            pltpu.SemaphoreType.DMA,
        ],
    )
    def k(table_hbm, src_hbm, dst_hbm, zeros_hbm, out_hbm,
          srcn_v, dst_v, dstb_v, rows_v, dstt_v, rowst_v, shared,
          gs0, gs1, ss0, ss1):
        c = jax.lax.axis_index("c")
        s = jax.lax.axis_index("s")
        base = c * half + s * EW2
        gsems = (gs0, gs1)
        ssems = (ss0, ss1)

        pltpu.sync_copy(src_hbm.at[pl.ds(base, EW2)], srcn_v)
        pltpu.sync_copy(dst_hbm.at[pl.ds(base, EW2)], dst_v)

        if nch > 1:
            @pl.loop(0, EW2 // 16)
            def _(t):
                sl = pl.ds(t * 16, 16)
                srcn_v[sl] = srcn_v[sl] * nch

        def gather_start(p, i):
            pltpu.async_copy(table_hbm.at[srcn_v.at[pl.ds(i * EBA, EBA)]],
                             rows_v.at[p], gsems[p])

        def gather_wait(p, i):
            pltpu.make_async_copy(table_hbm.at[srcn_v.at[pl.ds(i * EBA, EBA)]],
                                  rows_v.at[p], gsems[p]).wait()

        def load_dstb(p, i):
            for t in range(EBA // 16):
                sl = pl.ds(t * 16, 16)
                dstb_v[p, sl] = dst_v[pl.ds(i * EBA + t * 16, 16)]

        def scat_start(p):
            pltpu.async_copy(rows_v.at[p], shared.at[dstb_v.at[p]], ssems[p],
                             add=True)

        def scat_wait(p):
            pltpu.make_async_copy(rows_v.at[p], shared.at[dstb_v.at[p]],
                                  ssems[p]).wait()

        for q in range(nch):
            _ranged_copy(s, lambda o, L: zeros_hbm.at[pl.ds(0, L)],
                         lambda o, L: shared.at[pl.ds(o, L)])
            plsc.subcore_barrier()

            gather_start(0, 0)

            @pl.loop(0, NBLKF // 2)
            def _(j):
                for p in (0, 1):
                    i = 2 * j + p
                    gather_wait(p, i)
                    load_dstb(p, i)
                    scat_start(p)
                    if p == 1:
                        scat_wait(0)

                        @pl.when(j < NBLKF // 2 - 1)
                        def _():
                            gather_start(0, i + 1)
                    else:
                        @pl.when(j > 0)
                        def _():
                            scat_wait(1)
                        gather_start(1, i + 1)

            scat_wait(1)
            pltpu.sync_copy(table_hbm.at[srcn_v.at[pl.ds(TOFF, TAIL)]],
                            rowst_v)
            dstt_v[0, :] = dst_v[pl.ds(TOFF, TAIL)]
            pltpu.sync_copy(rowst_v, shared.at[dstt_v.at[0]], add=True)

            plsc.subcore_barrier()
            _ranged_copy(s, lambda o, L: shared.at[pl.ds(o, L)],
                         lambda o, L: out_hbm.at[c, q, pl.ds(o, L)])
            plsc.subcore_barrier()
            if q < nch - 1:
                @pl.loop(0, EW2 // 16)
                def _(t):
                    sl = pl.ds(t * 16, 16)
                    srcn_v[sl] = srcn_v[sl] + 1

    return k(table, src, dst, zeros)


def _head_sc_call(tid, pairs2, h3, h3w):
    BH = 32
    TW = T // 32
    NB = TW // BH

    @functools.partial(
        pl.kernel,
        out_type=jax.ShapeDtypeStruct((T, 16), jnp.float32),
        mesh=_SC_MESH,
        scratch_types=[
            pltpu.VMEM((TW,), jnp.int32),
            pltpu.VMEM((TW,), jnp.int32),
            pltpu.VMEM((BH, 128), jnp.int32),
            pltpu.VMEM((2, BH), jnp.int32),
            pltpu.VMEM((2, BH), jnp.int32),
            pltpu.VMEM((2, BH, H), jnp.float32),
            pltpu.VMEM((2, BH, H), jnp.float32),
            pltpu.VMEM((BH, 16), jnp.float32),
            pltpu.SemaphoreType.DMA,
            pltpu.SemaphoreType.DMA,
        ],
        compiler_params=_SC_CP,
    )
    def k(tid_hbm, pairs_hbm, h3_hbm, h3w_hbm, out_hbm,
          tid_v, ridx_v, prow_v, a_v, b_v, ra_v, rb_v, o16_v, rs0, rs1):
        c = jax.lax.axis_index("c")
        s = jax.lax.axis_index("s")
        base = (c * 16 + s) * TW
        rsems = (rs0, rs1)
        iota = jax.lax.iota(jnp.int32, 16)

        pltpu.sync_copy(tid_hbm.at[pl.ds(base, TW)], tid_v)

        @pl.loop(0, TW // 16)
        def _(t):
            sl = pl.ds(t * 16, 16)
            ridx_v[sl] = jax.lax.shift_right_logical(tid_v[sl], 6)

        def stage_a(p, i):
            pltpu.sync_copy(pairs_hbm.at[ridx_v.at[pl.ds(i * BH, BH)]],
                            prow_v)
            for g in range(BH // 16):
                sl = pl.ds(g * 16, 16)
                rows = iota + g * 16
                lane = (tid_v[pl.ds(i * BH + g * 16, 16)] & 63) * 2
                a_v[p, sl] = plsc.load_gather(prow_v, [rows, lane])
                b_v[p, sl] = plsc.load_gather(prow_v, [rows, lane + 1])
            pltpu.async_copy(h3_hbm.at[a_v.at[p]], ra_v.at[p], rsems[p])
            pltpu.async_copy(h3w_hbm.at[b_v.at[p]], rb_v.at[p], rsems[p])

        def rows_wait(p):
            pltpu.make_async_copy(h3_hbm.at[a_v.at[p]], ra_v.at[p],
                                  rsems[p]).wait()
            pltpu.make_async_copy(h3w_hbm.at[b_v.at[p]], rb_v.at[p],
                                  rsems[p]).wait()

        stage_a(0, 0)

        @pl.loop(0, NB // 2)
        def _(j):
            for p in (0, 1):
                i = 2 * j + p
                rows_wait(p)

                @pl.when(i < NB - 1)
                def _():
                    stage_a(p ^ 1, i + 1)

                def edge_dot(e, _):
                    acc = jnp.zeros((16,), jnp.float32)
                    for t in range(H // 16):
                        sl = pl.ds(t * 16, 16)
                        acc = acc + ra_v[p, e, sl] * rb_v[p, e, sl]
                    o16_v[e, :] = acc
                    return 0

                jax.lax.fori_loop(0, BH, edge_dot, 0)
                pltpu.sync_copy(o16_v,
                                out_hbm.at[pl.ds(base + i * BH, BH)])

    return k(tid, pairs2, h3, h3w)


def _tmp_agg(y, src, dst, nch):
    s = jax.ops.segment_sum(y[src], dst, num_segments=N)
    sq = s.reshape(N, nch, 128).transpose(1, 0, 2)
    parts = jnp.zeros((2, nch, NP, 128), jnp.float32).at[0, :, :N].set(sq)
    return parts


def kernel(x, edge_index, train_edge_id, W1, b1, W2, b2, W3, b3, W4, b4):
    src = edge_index[0]
    dst = edge_index[1]

    hist = _deg_call(dst)
    dinv, y0 = _prep_call(hist, x)

    w4r = W4.reshape(1, H)
    b1r = b1.reshape(1, H)
    b2r = b2.reshape(1, H)
    b3r = b3.reshape(1, H)

    parts1 = _agg_call(y0, src, dst, 1)
    y1, _ = _layer_call(parts1, y0, dinv, W1, b1r, w4r, last=False)

    parts2 = _agg_call(y1.reshape(N * 4, 128), src, dst, 4)
    y2, _ = _layer_call(parts2, y1, dinv, W2, b2r, w4r, last=False)

    parts3 = _agg_call(y2.reshape(N * 4, 128), src, dst, 4)
    h3, h3w = _layer_call(parts3, y2, dinv, W3, b3r, w4r, last=True)

    pairs2 = jnp.stack([src, dst], axis=1).reshape(E // 64, 128)
    p16 = _head_sc_call(train_edge_id, pairs2, h3, h3w)
    return _headsum_call(p16, b4.reshape(1, 1))


# speedup vs baseline: 2.2657x; 1.0008x over previous
import dataclasses
import functools

import jax
import jax.numpy as jnp
from jax.experimental import pallas as pl
from jax.experimental.pallas import tpu as pltpu
from jax.experimental.pallas import tpu_sc as plsc

N = 10000
NP = 10000
E = 320000
D = 128
H = 512
T = 65536


def _prep_body(hist_ref, x_ref, dinv_ref, y0_ref):
    deg = 1.0 + hist_ref[0, :N, 0:1] + hist_ref[1, :N, 0:1]
    dinv = jax.lax.rsqrt(deg)
    dinv_ref[...] = dinv
    y0_ref[...] = x_ref[...] * dinv


def _prep_call(histT, x):
    return pl.pallas_call(
        _prep_body,
        out_shape=(
            jax.ShapeDtypeStruct((N, 1), jnp.float32),
            jax.ShapeDtypeStruct((N, D), jnp.float32),
        ),

    )(histT, x)


def _layer_body(nch, last, parts_ref, yprev_ref, dinv_ref, w_ref, b_ref,
                w4r_ref, o1_ref, o2_ref):
    bn = yprev_ref.shape[0]
    dinv = dinv_ref[...]
    acc = jnp.zeros((bn, H), jnp.float32)
    for q in range(nch):
        zq = (parts_ref[0, q] + parts_ref[1, q]
              + yprev_ref[:, q * 128:(q + 1) * 128]) * dinv
        acc = acc + jnp.dot(zq, w_ref[q * 128:(q + 1) * 128, :],
                            preferred_element_type=jnp.float32)
    h = jnp.tanh(acc + b_ref[...])
    if last:
        o1_ref[...] = h
        o2_ref[...] = h * w4r_ref[...]
    else:
        o1_ref[...] = h * dinv
        o2_ref[...] = h


def _layer_call(parts, yprev, dinv, w, b2d, w4r, last):
    nch = parts.shape[1]
    fin = nch * 128
    bn = 1000
    grid = (N // bn,)
    body = functools.partial(_layer_body, nch, last)
    return pl.pallas_call(
        body,
        grid=grid,
        in_specs=[
            pl.BlockSpec((2, nch, bn, 128), lambda i: (0, 0, i, 0)),
            pl.BlockSpec((bn, fin), lambda i: (i, 0)),
            pl.BlockSpec((bn, 1), lambda i: (i, 0)),
            pl.BlockSpec((fin, H), lambda i: (0, 0)),
            pl.BlockSpec((1, H), lambda i: (0, 0)),
            pl.BlockSpec((1, H), lambda i: (0, 0)),
        ],
        out_specs=(
            pl.BlockSpec((bn, H), lambda i: (i, 0)),
            pl.BlockSpec((bn, H), lambda i: (i, 0)),
        ),
        out_shape=(
            jax.ShapeDtypeStruct((N, H), jnp.float32),
            jax.ShapeDtypeStruct((N, H), jnp.float32),
        ),

    )(parts, yprev, dinv, w, b2d, w4r)


def _headsum_body(p16_ref, b4_ref, out_ref):
    out_ref[...] = jnp.sum(p16_ref[...], axis=1, keepdims=True) + b4_ref[...]


def _headsum_call(p16, b4_2d):
    bt = 2048
    return pl.pallas_call(
        _headsum_body,
        grid=(T // bt,),
        in_specs=[
            pl.BlockSpec((bt, 16), lambda i: (i, 0)),
            pl.BlockSpec((1, 1), lambda i: (0, 0)),
        ],
        out_specs=pl.BlockSpec((bt, 1), lambda i: (i, 0)),
        out_shape=jax.ShapeDtypeStruct((T, 1), jnp.float32),

    )(p16, b4_2d)


_SC_MESH = plsc.VectorSubcoreMesh(core_axis_name="c", subcore_axis_name="s")

_SC_CP = pltpu.CompilerParams()
if "needs_layout_passes" in pltpu.CompilerParams.__dataclass_fields__:
    _SC_CP = dataclasses.replace(_SC_CP, needs_layout_passes=False)

NR0 = 624
NRL = NP - 15 * NR0


def _ranged_copy(s, mk_src, mk_dst):
    @pl.when(s < 15)
    def _():
        pltpu.sync_copy(mk_src(s * NR0, NR0), mk_dst(s * NR0, NR0))

    @pl.when(s == 15)
    def _():
        pltpu.sync_copy(mk_src(15 * NR0, NRL), mk_dst(15 * NR0, NRL))
EB = 80
EW = E // 32


def _deg_call(dst):
    RW = 128
    zeros = jnp.zeros((NRL, RW), jnp.float32)
    ones = jnp.ones((EB, RW), jnp.float32)

    @functools.partial(
        pl.kernel,
        out_type=jax.ShapeDtypeStruct((2, NP, RW), jnp.float32),
        mesh=_SC_MESH,
        scratch_types=[
            pltpu.VMEM((1, EB), jnp.int32),
            pltpu.VMEM((EB, RW), jnp.float32),
            pltpu.VMEM_SHARED((NP, RW), jnp.float32),
        ],
    )
    def k(dst_hbm, zeros_hbm, ones_hbm, out_hbm, dst_v, ones_v, shared):
        c = jax.lax.axis_index("c")
        s = jax.lax.axis_index("s")
        _ranged_copy(s, lambda o, L: zeros_hbm.at[pl.ds(0, L)],
                     lambda o, L: shared.at[pl.ds(o, L)])
        pltpu.sync_copy(ones_hbm, ones_v)
        plsc.subcore_barrier()
        base = (c * 16 + s) * EW

        @pl.loop(0, EW // EB)
        def _(i):
            pltpu.sync_copy(dst_hbm.at[pl.ds(base + i * EB, EB)], dst_v.at[0])
            pltpu.sync_copy(ones_v, shared.at[dst_v.at[0]], add=True)

        plsc.subcore_barrier()
        _ranged_copy(s, lambda o, L: shared.at[pl.ds(o, L)],
                     lambda o, L: out_hbm.at[c, pl.ds(o, L)])

    return k(dst, zeros, ones)


EW2 = E // 32
EBA = 96
NBLKF = EW2 // EBA
TAIL = EW2 - NBLKF * EBA
TOFF = NBLKF * EBA


def _agg_call(table, src, dst, nch):
    zeros = jnp.zeros((NRL, 128), jnp.float32)
    half = E // 2

    @functools.partial(
        pl.kernel,
        out_type=jax.ShapeDtypeStruct((2, nch, NP, 128), jnp.float32),
        mesh=_SC_MESH,
        scratch_types=[
            pltpu.VMEM((EW2,), jnp.int32),
            pltpu.VMEM((EW2,), jnp.int32),
            pltpu.VMEM((2, EBA), jnp.int32),
            pltpu.VMEM((2, EBA, 128), jnp.float32),
            pltpu.VMEM((1, TAIL), jnp.int32),
            pltpu.VMEM((TAIL, 128), jnp.float32),
            pltpu.VMEM_SHARED((NP, 128), jnp.float32),
            pltpu.SemaphoreType.DMA,
            pltpu.SemaphoreType.DMA,
            pltpu.SemaphoreType.DMA,
            pltpu.SemaphoreType.DMA,
        ],
    )
    def k(table_hbm, src_hbm, dst_hbm, zeros_hbm, out_hbm,
          srcn_v, dst_v, dstb_v, rows_v, dstt_v, rowst_v, shared,
          gs0, gs1, ss0, ss1):
        c = jax.lax.axis_index("c")
        s = jax.lax.axis_index("s")
        base = c * half + s * EW2
        gsems = (gs0, gs1)
        ssems = (ss0, ss1)

        pltpu.sync_copy(src_hbm.at[pl.ds(base, EW2)], srcn_v)
        pltpu.sync_copy(dst_hbm.at[pl.ds(base, EW2)], dst_v)

        if nch > 1:
            @pl.loop(0, EW2 // 16)
            def _(t):
                sl = pl.ds(t * 16, 16)
                srcn_v[sl] = srcn_v[sl] * nch

        def gather_start(p, i):
            pltpu.async_copy(table_hbm.at[srcn_v.at[pl.ds(i * EBA, EBA)]],
                             rows_v.at[p], gsems[p])

        def gather_wait(p, i):
            pltpu.make_async_copy(table_hbm.at[srcn_v.at[pl.ds(i * EBA, EBA)]],
                                  rows_v.at[p], gsems[p]).wait()

        def load_dstb(p, i):
            for t in range(EBA // 16):
                sl = pl.ds(t * 16, 16)
                dstb_v[p, sl] = dst_v[pl.ds(i * EBA + t * 16, 16)]

        def scat_start(p):
            pltpu.async_copy(rows_v.at[p], shared.at[dstb_v.at[p]], ssems[p],
                             add=True)

        def scat_wait(p):
            pltpu.make_async_copy(rows_v.at[p], shared.at[dstb_v.at[p]],
                                  ssems[p]).wait()

        for q in range(nch):
            _ranged_copy(s, lambda o, L: zeros_hbm.at[pl.ds(0, L)],
                         lambda o, L: shared.at[pl.ds(o, L)])
            plsc.subcore_barrier()

            gather_start(0, 0)

            @pl.loop(0, NBLKF // 2)
            def _(j):
                for p in (0, 1):
                    i = 2 * j + p
                    gather_wait(p, i)
                    load_dstb(p, i)
                    scat_start(p)
                    if p == 1:
                        scat_wait(0)

                        @pl.when(j < NBLKF // 2 - 1)
                        def _():
                            gather_start(0, i + 1)
                    else:
                        @pl.when(j > 0)
                        def _():
                            scat_wait(1)
                        gather_start(1, i + 1)

            scat_wait(1)
            pltpu.sync_copy(table_hbm.at[srcn_v.at[pl.ds(TOFF, TAIL)]],
                            rowst_v)
            dstt_v[0, :] = dst_v[pl.ds(TOFF, TAIL)]
            pltpu.sync_copy(rowst_v, shared.at[dstt_v.at[0]], add=True)

            plsc.subcore_barrier()
            _ranged_copy(s, lambda o, L: shared.at[pl.ds(o, L)],
                         lambda o, L: out_hbm.at[c, q, pl.ds(o, L)])
            plsc.subcore_barrier()
            if q < nch - 1:
                @pl.loop(0, EW2 // 16)
                def _(t):
                    sl = pl.ds(t * 16, 16)
                    srcn_v[sl] = srcn_v[sl] + 1

    return k(table, src, dst, zeros)


def _head_sc_call(tid, pairs2, h3, h3w):
    BH = 32
    TW = T // 32
    NB = TW // BH

    @functools.partial(
        pl.kernel,
        out_type=jax.ShapeDtypeStruct((T, 16), jnp.float32),
        mesh=_SC_MESH,
        scratch_types=[
            pltpu.VMEM((TW,), jnp.int32),
            pltpu.VMEM((TW,), jnp.int32),
            pltpu.VMEM((BH, 128), jnp.int32),
            pltpu.VMEM((2, BH), jnp.int32),
            pltpu.VMEM((2, BH), jnp.int32),
            pltpu.VMEM((2, BH, H), jnp.float32),
            pltpu.VMEM((2, BH, H), jnp.float32),
            pltpu.VMEM((BH, 16), jnp.float32),
            pltpu.SemaphoreType.DMA,
            pltpu.SemaphoreType.DMA,
        ],
        compiler_params=_SC_CP,
    )
    def k(tid_hbm, pairs_hbm, h3_hbm, h3w_hbm, out_hbm,
          tid_v, ridx_v, prow_v, a_v, b_v, ra_v, rb_v, o16_v, rs0, rs1):
        c = jax.lax.axis_index("c")
        s = jax.lax.axis_index("s")
        base = (c * 16 + s) * TW
        rsems = (rs0, rs1)
        iota = jax.lax.iota(jnp.int32, 16)

        pltpu.sync_copy(tid_hbm.at[pl.ds(base, TW)], tid_v)

        @pl.loop(0, TW // 16)
        def _(t):
            sl = pl.ds(t * 16, 16)
            ridx_v[sl] = jax.lax.shift_right_logical(tid_v[sl], 6)

        def stage_a(p, i):
            pltpu.sync_copy(pairs_hbm.at[ridx_v.at[pl.ds(i * BH, BH)]],
                            prow_v)
            for g in range(BH // 16):
                sl = pl.ds(g * 16, 16)
                rows = iota + g * 16
                lane = (tid_v[pl.ds(i * BH + g * 16, 16)] & 63) * 2
                a_v[p, sl] = plsc.load_gather(prow_v, [rows, lane])
                b_v[p, sl] = plsc.load_gather(prow_v, [rows, lane + 1])
            pltpu.async_copy(h3_hbm.at[a_v.at[p]], ra_v.at[p], rsems[p])
            pltpu.async_copy(h3w_hbm.at[b_v.at[p]], rb_v.at[p], rsems[p])

        def rows_wait(p):
            pltpu.make_async_copy(h3_hbm.at[a_v.at[p]], ra_v.at[p],
                                  rsems[p]).wait()
            pltpu.make_async_copy(h3w_hbm.at[b_v.at[p]], rb_v.at[p],
                                  rsems[p]).wait()

        stage_a(0, 0)

        @pl.loop(0, NB // 2)
        def _(j):
            for p in (0, 1):
                i = 2 * j + p
                rows_wait(p)

                @pl.when(i < NB - 1)
                def _():
                    stage_a(p ^ 1, i + 1)

                def edge_dot(e, _):
                    acc = jnp.zeros((16,), jnp.float32)
                    for t in range(H // 16):
                        sl = pl.ds(t * 16, 16)
                        acc = acc + ra_v[p, e, sl] * rb_v[p, e, sl]
                    o16_v[e, :] = acc
                    return 0

                jax.lax.fori_loop(0, BH, edge_dot, 0)
                pltpu.sync_copy(o16_v,
                                out_hbm.at[pl.ds(base + i * BH, BH)])

    return k(tid, pairs2, h3, h3w)


def kernel(x, edge_index, train_edge_id, W1, b1, W2, b2, W3, b3, W4, b4):
    src = edge_index[0]
    dst = edge_index[1]

    hist = _deg_call(dst)
    dinv, y0 = _prep_call(hist, x)

    w4r = W4.reshape(1, H)
    b1r = b1.reshape(1, H)
    b2r = b2.reshape(1, H)
    b3r = b3.reshape(1, H)

    parts1 = _agg_call(y0, src, dst, 1)
    y1, _ = _layer_call(parts1, y0, dinv, W1, b1r, w4r, last=False)

    parts2 = _agg_call(y1.reshape(N * 4, 128), src, dst, 4)
    y2, _ = _layer_call(parts2, y1, dinv, W2, b2r, w4r, last=False)

    parts3 = _agg_call(y2.reshape(N * 4, 128), src, dst, 4)
    h3, h3w = _layer_call(parts3, y2, dinv, W3, b3r, w4r, last=True)

    pairs2 = jnp.stack([src, dst], axis=1).reshape(E // 64, 128)
    p16 = _head_sc_call(train_edge_id, pairs2, h3, h3w)
    return _headsum_call(p16, b4.reshape(1, 1))
